# baseline (speedup 1.0000x reference)
import functools

import jax
import jax.numpy as jnp
from jax import lax
from jax.experimental import pallas as pl
from jax.experimental.pallas import tpu as pltpu
from jax.experimental.pallas import tpu_sc as plsc

_N = 200
_E = 12800
_NP = 256
_LANES = 16
_SUBC = 16
_ROWS = _NP // _SUBC


def _sc_build_adj(ei1, ea1, ei2, ea2):
    mesh = plsc.VectorSubcoreMesh(core_axis_name="c", subcore_axis_name="s")

    @functools.partial(
        pl.kernel,
        out_type=jax.ShapeDtypeStruct((2, _NP, _NP), jnp.float32),
        mesh=mesh,
        scratch_types=[
            pltpu.VMEM((_E,), jnp.int32),
            pltpu.VMEM((_E,), jnp.int32),
            pltpu.VMEM((_E,), jnp.float32),
            pltpu.VMEM((_ROWS, _NP), jnp.float32),
        ],
        compiler_params=pltpu.CompilerParams(needs_layout_passes=False),
    )
    def build(ei1_h, ea1_h, ei2_h, ea2_h, out_h, src_v, dst_v, ew_v, acc_v):
        c = lax.axis_index("c")
        s = lax.axis_index("s")
        base = s * _ROWS
        zeros = jnp.zeros((_LANES,), jnp.float32)

        def body(ei_h, ea_h, g):
            pltpu.sync_copy(ei_h.at[0], src_v)
            pltpu.sync_copy(ei_h.at[1], dst_v)
            pltpu.sync_copy(ea_h, ew_v)

            def zstep(r, carry):
                for k in range(_NP // _LANES):
                    acc_v[r, pl.ds(k * _LANES, _LANES)] = zeros
                return carry
            lax.fori_loop(0, _ROWS, zstep, 0)

            def estep(i, carry):
                e0 = i * _LANES
                s16 = src_v[pl.ds(e0, _LANES)]
                d16 = dst_v[pl.ds(e0, _LANES)]
                w16 = ew_v[pl.ds(e0, _LANES)]
                w16 = jnp.where(s16 == d16, 0.0, w16)
                rel = d16 - base
                inr = (rel >= 0) & (rel < _ROWS)
                relc = jnp.where(inr, rel, 0)
                plsc.addupdate_scatter(acc_v, [relc, s16], w16, mask=inr)
                return carry
            lax.fori_loop(0, _E // _LANES, estep, 0)

            pltpu.sync_copy(acc_v, out_h.at[g, pl.ds(base, _ROWS)])

        @pl.when(c == 0)
        def _():
            body(ei1_h, ea1_h, 0)

        @pl.when(c == 1)
        def _():
            body(ei2_h, ea2_h, 1)

    return build(ei1, ea1, ei2, ea2)


def _tc_forward(adj, x1, x2, gc1_W, gc1_b, gc4_W, gc4_b, cls_W1, cls_b1,
                cls_W2, cls_b2):
    pad_n = _NP - _N

    def body(a_r, x1_r, x2_r, w1_r, b1_r, w4_r, b4_r, cw1_r, cb1_r, cw2_r,
             cb2_r, out_r):
        def split(v):
            hi = v.astype(jnp.bfloat16)
            return hi, (v - hi.astype(jnp.float32)).astype(jnp.bfloat16)

        def make_l(A):
            deg = jnp.sum(A, axis=0)
            safe = jnp.where(deg > 0, deg, 1.0)
            dis = jnp.where(deg > 0, 1.0 / jnp.sqrt(safe), 0.0)
            return split(-(dis[:, None] * A * dis[None, :]))

        def prop(L, x):
            l_hi, l_lo = L
            x_hi, x_lo = split(x)
            t = jnp.dot(l_hi, x_hi, preferred_element_type=jnp.float32)
            t = t + jnp.dot(l_hi, x_lo, preferred_element_type=jnp.float32)
            return t + jnp.dot(l_lo, x_hi, preferred_element_type=jnp.float32)

        def cheb(x, L, w_r, b):
            out = jnp.dot(x, w_r[0], preferred_element_type=jnp.float32)
            t1 = prop(L, x)
            out = out + jnp.dot(t1, w_r[1], preferred_element_type=jnp.float32)
            t2 = 2.0 * prop(L, t1) - x
            out = out + jnp.dot(t2, w_r[2], preferred_element_type=jnp.float32)
            return out + b

        def tower(x, L, b1, b4):
            h = jnp.maximum(cheb(x, L, w1_r, b1), 0.0)
            return jnp.maximum(cheb(h, L, w4_r, b4), 0.0)

        xpad = jnp.zeros((pad_n, x1_r.shape[1]), jnp.float32)
        x1p = jnp.concatenate([x1_r[...], xpad], axis=0)
        x2p = jnp.concatenate([x2_r[...], xpad], axis=0)
        b1 = b1_r[...]
        b4 = b4_r[...]
        h1 = tower(x1p, make_l(a_r[0]), b1, b4)
        h2 = tower(x2p, make_l(a_r[1]), b1, b4)
        prod = h1 * h2

        cw1 = jnp.pad(cw1_r[...], ((0, pad_n), (0, 28)))
        cb1 = jnp.pad(cb1_r[...], (0, 28))
        cw2 = jnp.pad(cw2_r[...], ((0, 28), (0, 0)))
        hid = lax.dot_general(prod, cw1, (((0,), (0,)), ((), ())),
                              preferred_element_type=jnp.float32)
        hid = jnp.maximum(hid + cb1, 0.0)
        out_r[...] = jnp.dot(hid, cw2,
                             preferred_element_type=jnp.float32) + cb2_r[...]

    return pl.pallas_call(
        body,
        out_shape=jax.ShapeDtypeStruct((_NP, 1), jnp.float32),
    )(adj, x1, x2, gc1_W, gc1_b, gc4_W, gc4_b, cls_W1, cls_b1, cls_W2, cls_b2)


def kernel(x1, edge_index1, edge_attr1, x2, edge_index2, edge_attr2, gc1_W,
           gc1_b, gc4_W, gc4_b, cls_W1, cls_b1, cls_W2, cls_b2):
    adj = _sc_build_adj(edge_index1, edge_attr1, edge_index2, edge_attr2)
    return _tc_forward(adj, x1, x2, gc1_W, gc1_b, gc4_W, gc4_b, cls_W1,
                       cls_b1, cls_W2, cls_b2)
